# baseline (speedup 1.0000x reference)
import jax
import jax.numpy as jnp
from jax import lax
from jax.experimental import pallas as pl
from jax.experimental.pallas import tpu as pltpu
from jax.experimental.pallas import tpu_sc as plsc

_BLOCK_B = 8
_ROWS = 43008
_W = 224
_NWORKERS = 32
_ROWS_PER_WORKER = _ROWS // _NWORKERS
_CHUNKS = 8
_CHUNK_ROWS = _ROWS_PER_WORKER // _CHUNKS


def _noising_kernel(t_ref, betas_bar_ref, x_ref, noise_ref, out_ref):
    t = t_ref[0]
    beta = betas_bar_ref[t, 0]
    sa = jnp.sqrt(1.0 - beta)
    sb = jnp.sqrt(beta)
    out_ref[...] = sa * x_ref[...] + sb * noise_ref[...]


def _sc_copy_kernel(src_hbm, dst_hbm, buf0, buf1, in_s0, in_s1, out_s0, out_s1):
    wid = lax.axis_index("s") * 2 + lax.axis_index("c")
    base = wid * _ROWS_PER_WORKER
    bufs = (buf0, buf1)
    in_sems = (in_s0, in_s1)
    out_sems = (out_s0, out_s1)

    def in_copy(k):
        return pltpu.make_async_copy(
            src_hbm.at[pl.ds(base + k * _CHUNK_ROWS, _CHUNK_ROWS)],
            bufs[k % 2], in_sems[k % 2])

    def out_copy(k):
        return pltpu.make_async_copy(
            bufs[k % 2],
            dst_hbm.at[pl.ds(base + k * _CHUNK_ROWS, _CHUNK_ROWS)],
            out_sems[k % 2])

    in_copy(0).start()
    for k in range(_CHUNKS):
        in_copy(k).wait()
        out_copy(k).start()
        if k + 1 < _CHUNKS:
            if k >= 1:
                out_copy(k - 1).wait()
            in_copy(k + 1).start()
    out_copy(_CHUNKS - 2).wait()
    out_copy(_CHUNKS - 1).wait()


def kernel(x, t, betas_bar, noise):
    t_arr = jnp.asarray(t, dtype=jnp.int32).reshape((1,))
    b, c, h, w = x.shape

    sc_copy = pl.kernel(
        _sc_copy_kernel,
        out_type=jax.ShapeDtypeStruct((_ROWS, _W), noise.dtype),
        mesh=plsc.VectorSubcoreMesh(core_axis_name="c", subcore_axis_name="s"),
        scratch_types=[
            pltpu.VMEM((_CHUNK_ROWS, _W), noise.dtype),
            pltpu.VMEM((_CHUNK_ROWS, _W), noise.dtype),
            pltpu.SemaphoreType.DMA,
            pltpu.SemaphoreType.DMA,
            pltpu.SemaphoreType.DMA,
            pltpu.SemaphoreType.DMA,
        ],
        compiler_params=pltpu.CompilerParams(use_tc_tiling_on_sc=True),
    )
    noise_out = sc_copy(noise.reshape(_ROWS, _W)).reshape(x.shape)

    blk = (_BLOCK_B, c, h, w)
    noised = pl.pallas_call(
        _noising_kernel,
        grid=(b // _BLOCK_B,),
        in_specs=[
            pl.BlockSpec(memory_space=pltpu.SMEM),
            pl.BlockSpec(memory_space=pltpu.SMEM),
            pl.BlockSpec(blk, lambda i: (i, 0, 0, 0)),
            pl.BlockSpec(blk, lambda i: (i, 0, 0, 0)),
        ],
        out_specs=pl.BlockSpec(blk, lambda i: (i, 0, 0, 0)),
        out_shape=jax.ShapeDtypeStruct(x.shape, x.dtype),
    )(t_arr, betas_bar, x, noise)

    return noised, noise_out


# speedup vs baseline: 1.3425x; 1.3425x over previous
import jax
import jax.numpy as jnp
from jax import lax
from jax.experimental import pallas as pl
from jax.experimental.pallas import tpu as pltpu

_ROWS = 43008
_W = 224
_NCHUNK = 8
_CR = _ROWS // _NCHUNK
_D = 4
_S = 4
_SR = _CR // _S


def _ring_kernel(t_ref, betas_bar_ref, x_hbm, n_hbm, y_hbm, ny_hbm,
                 xbuf, nbuf, in_sems, out_sems):
    i = pl.program_id(0)

    def in_cps(k, slot):
        cps = []
        for s in range(_S):
            cps.append(pltpu.make_async_copy(
                x_hbm.at[pl.ds(k * _CR + s * _SR, _SR)],
                xbuf.at[slot, pl.ds(s * _SR, _SR)], in_sems.at[slot, 2 * s]))
            cps.append(pltpu.make_async_copy(
                n_hbm.at[pl.ds(k * _CR + s * _SR, _SR)],
                nbuf.at[slot, pl.ds(s * _SR, _SR)], in_sems.at[slot, 2 * s + 1]))
        return cps

    def out_cps(k, slot):
        cps = []
        for s in range(_S):
            cps.append(pltpu.make_async_copy(
                xbuf.at[slot, pl.ds(s * _SR, _SR)],
                y_hbm.at[pl.ds(k * _CR + s * _SR, _SR)], out_sems.at[slot, 2 * s]))
            cps.append(pltpu.make_async_copy(
                nbuf.at[slot, pl.ds(s * _SR, _SR)],
                ny_hbm.at[pl.ds(k * _CR + s * _SR, _SR)], out_sems.at[slot, 2 * s + 1]))
        return cps

    @pl.when(i == 0)
    def _prologue():
        for k in range(_D - 1):
            for c in in_cps(k, k):
                c.start()

    t = t_ref[0]
    beta = betas_bar_ref[t, 0]
    sa = jnp.sqrt(1.0 - beta)
    sb = jnp.sqrt(beta)

    for slot in range(_D):
        @pl.when(lax.rem(i, _D) == slot)
        def _step(slot=slot):
            for c in in_cps(i, slot):
                c.wait()
            xbuf[slot] = sa * xbuf[slot] + sb * nbuf[slot]
            for c in out_cps(i, slot):
                c.start()

    j = i + _D - 1

    @pl.when(j < _NCHUNK)
    def _refill():
        @pl.when(i >= 1)
        def _drain_prev():
            for slot in range(_D):
                @pl.when(lax.rem(i - 1, _D) == slot)
                def _w(slot=slot):
                    for c in out_cps(i - 1, slot):
                        c.wait()

        for slot in range(_D):
            @pl.when(lax.rem(j, _D) == slot)
            def _s(slot=slot):
                for c in in_cps(j, slot):
                    c.start()

    @pl.when(i == _NCHUNK - 1)
    def _epilogue():
        for k in range(_NCHUNK - _D, _NCHUNK):
            for c in out_cps(k, k % _D):
                c.wait()


def kernel(x, t, betas_bar, noise):
    t_arr = jnp.asarray(t, dtype=jnp.int32).reshape((1,))
    x2 = x.reshape(_ROWS, _W)
    n2 = noise.reshape(_ROWS, _W)
    noised, noise_out = pl.pallas_call(
        _ring_kernel,
        grid=(_NCHUNK,),
        in_specs=[
            pl.BlockSpec(memory_space=pltpu.SMEM),
            pl.BlockSpec(memory_space=pltpu.SMEM),
            pl.BlockSpec(memory_space=pltpu.MemorySpace.HBM),
            pl.BlockSpec(memory_space=pltpu.MemorySpace.HBM),
        ],
        out_specs=[
            pl.BlockSpec(memory_space=pltpu.MemorySpace.HBM),
            pl.BlockSpec(memory_space=pltpu.MemorySpace.HBM),
        ],
        out_shape=[
            jax.ShapeDtypeStruct((_ROWS, _W), x.dtype),
            jax.ShapeDtypeStruct((_ROWS, _W), x.dtype),
        ],
        scratch_shapes=[
            pltpu.VMEM((_D, _CR, _W), x.dtype),
            pltpu.VMEM((_D, _CR, _W), x.dtype),
            pltpu.SemaphoreType.DMA((_D, 2 * _S)),
            pltpu.SemaphoreType.DMA((_D, 2 * _S)),
        ],
        compiler_params=pltpu.CompilerParams(vmem_limit_bytes=56 * 1024 * 1024),
    )(t_arr, betas_bar, x2, n2)
    return noised.reshape(x.shape), noise_out.reshape(x.shape)


# speedup vs baseline: 1.4109x; 1.0510x over previous
import jax
import jax.numpy as jnp
from jax.experimental import pallas as pl
from jax.experimental.pallas import tpu as pltpu

_ROWS = 43008
_W = 224
_GRID = 8
_BR = _ROWS // _GRID


def _noising_kernel(t_ref, betas_bar_ref, x_ref, noise_ref, out_ref, noise_out_ref):
    t = t_ref[0]
    beta = betas_bar_ref[t, 0]
    sa = jnp.sqrt(1.0 - beta)
    sb = jnp.sqrt(beta)
    n = noise_ref[...]
    out_ref[...] = sa * x_ref[...] + sb * n
    noise_out_ref[...] = n


def kernel(x, t, betas_bar, noise):
    t_arr = jnp.asarray(t, dtype=jnp.int32).reshape((1,))
    x2 = x.reshape(_ROWS, _W)
    n2 = noise.reshape(_ROWS, _W)
    blk = (_BR, _W)
    noised, noise_out = pl.pallas_call(
        _noising_kernel,
        grid=(_GRID,),
        in_specs=[
            pl.BlockSpec(memory_space=pltpu.SMEM),
            pl.BlockSpec(memory_space=pltpu.SMEM),
            pl.BlockSpec(blk, lambda i: (i, 0)),
            pl.BlockSpec(blk, lambda i: (i, 0)),
        ],
        out_specs=[
            pl.BlockSpec(blk, lambda i: (i, 0)),
            pl.BlockSpec(blk, lambda i: (i, 0)),
        ],
        out_shape=[
            jax.ShapeDtypeStruct((_ROWS, _W), x.dtype),
            jax.ShapeDtypeStruct((_ROWS, _W), x.dtype),
        ],
    )(t_arr, betas_bar, x2, n2)
    return noised.reshape(x.shape), noise_out.reshape(x.shape)


# speedup vs baseline: 1.4598x; 1.0346x over previous
import jax
import jax.numpy as jnp
from jax.experimental import pallas as pl
from jax.experimental.pallas import tpu as pltpu

_ROWS = 43008
_W = 224
_GRID = 4
_BR = _ROWS // _GRID
_BW = 128


def _noising_kernel(t_ref, betas_bar_ref, x_ref, noise_ref, out_ref, noise_out_ref):
    t = t_ref[0]
    beta = betas_bar_ref[t, 0]
    sa = jnp.sqrt(1.0 - beta)
    sb = jnp.sqrt(beta)
    n = noise_ref[...]
    out_ref[...] = sa * x_ref[...] + sb * n
    noise_out_ref[...] = n


def kernel(x, t, betas_bar, noise):
    t_arr = jnp.asarray(t, dtype=jnp.int32).reshape((1,))
    x2 = x.reshape(_ROWS, _W)
    n2 = noise.reshape(_ROWS, _W)
    blk = (_BR, _BW)
    noised, noise_out = pl.pallas_call(
        _noising_kernel,
        grid=(_GRID, 2),
        in_specs=[
            pl.BlockSpec(memory_space=pltpu.SMEM),
            pl.BlockSpec(memory_space=pltpu.SMEM),
            pl.BlockSpec(blk, lambda i, j: (i, j)),
            pl.BlockSpec(blk, lambda i, j: (i, j)),
        ],
        out_specs=[
            pl.BlockSpec(blk, lambda i, j: (i, j)),
            pl.BlockSpec(blk, lambda i, j: (i, j)),
        ],
        out_shape=[
            jax.ShapeDtypeStruct((_ROWS, _W), x.dtype),
            jax.ShapeDtypeStruct((_ROWS, _W), x.dtype),
        ],
    )(t_arr, betas_bar, x2, n2)
    return noised.reshape(x.shape), noise_out.reshape(x.shape)
